# Initial kernel scaffold; baseline (speedup 1.0000x reference)
import jax
import jax.numpy as jnp
from jax.experimental import pallas as pl


def kernel(x, table):
    raise NotImplementedError("write your pallas kernel here")


# speedup vs baseline: 1.5588x; 1.5588x over previous
import functools
import math

import jax
import jax.numpy as jnp
from jax import lax
from jax.experimental import pallas as pl
from jax.experimental.pallas import tpu as pltpu
from jax.experimental.pallas import tpu_sc as plsc

_VOCAB = 100000
_D = 512
_SEQ = 8192

_NC = 2
_NS = 16
_NW = _NC * _NS
_B_PER_W = _SEQ // _NW

_CH = 64
_NCH = _B_PER_W // _CH

_mesh = plsc.VectorSubcoreMesh(core_axis_name="c", subcore_axis_name="s")


@functools.partial(
    pl.kernel,
    mesh=_mesh,
    out_type=jax.ShapeDtypeStruct((_SEQ, _D), jnp.float32),
    scratch_types=[
        pltpu.VMEM((_NCH, _CH), jnp.int32),
        pltpu.VMEM((_CH, _D), jnp.float32),
        pltpu.VMEM((_CH, _D), jnp.float32),
        pltpu.SemaphoreType.DMA,
        pltpu.SemaphoreType.DMA,
        pltpu.SemaphoreType.DMA,
        pltpu.SemaphoreType.DMA,
    ],
)
def _sc_gather(table_hbm, idx_hbm, out_hbm, idx_v, buf0, buf1,
               gsem0, gsem1, wsem0, wsem1):
    wid = lax.axis_index("s") * _NC + lax.axis_index("c")
    base = wid * _B_PER_W
    pltpu.sync_copy(idx_hbm.at[wid], idx_v)
    bufs = (buf0, buf1)
    gsems = (gsem0, gsem1)
    wsems = (wsem0, wsem1)

    def start_gather(c):
        return pltpu.async_copy(table_hbm.at[idx_v.at[c]], bufs[c % 2],
                                gsems[c % 2])

    g = [start_gather(0), start_gather(1)]
    for c in range(_NCH):
        b = c % 2
        g[b].wait()
        w = pltpu.async_copy(bufs[b], out_hbm.at[pl.ds(base + c * _CH, _CH)],
                             wsems[b])
        if c + 2 < _NCH:
            w.wait()
            g[b] = start_gather(c + 2)
        else:
            w.wait()


_BLK = 512
_LOG1E4_2_OVER_D = 2.0 * math.log(10000.0) / _D
_NJ2 = 2 * ((_D - 1) // 2)


def _add_pos_body(emb_ref, out_ref):
    i = pl.program_id(0)
    row = (jnp.float32(i * _BLK)
           + lax.broadcasted_iota(jnp.int32, (_BLK, _D), 0).astype(jnp.float32))
    col = lax.broadcasted_iota(jnp.int32, (_BLK, _D), 1)
    j = jnp.floor_divide(col, 2).astype(jnp.float32)
    denom = jnp.exp(j * _LOG1E4_2_OVER_D)
    angle = row / denom
    pos = jnp.where(col % 2 == 0, jnp.sin(angle), jnp.cos(angle))
    pos = jnp.where(col >= _NJ2, 0.0, pos)
    out_ref[...] = emb_ref[...] + pos


def _add_pos(emb):
    return pl.pallas_call(
        _add_pos_body,
        grid=(_SEQ // _BLK,),
        in_specs=[pl.BlockSpec((_BLK, _D), lambda i: (i, 0))],
        out_specs=pl.BlockSpec((_BLK, _D), lambda i: (i, 0)),
        out_shape=jax.ShapeDtypeStruct((_SEQ, _D), jnp.float32),
    )(emb)


def kernel(x, table):
    idx = x.astype(jnp.int32).reshape(_NW, _NCH, _CH)
    emb = _sc_gather(table, idx)
    return _add_pos(emb)


# speedup vs baseline: 1.6343x; 1.0484x over previous
import functools
import math

import jax
import jax.numpy as jnp
from jax import lax
from jax.experimental import pallas as pl
from jax.experimental.pallas import tpu as pltpu
from jax.experimental.pallas import tpu_sc as plsc

_VOCAB = 100000
_D = 512
_SEQ = 8192

_NC = 2
_NS = 16
_NW = _NC * _NS
_B_PER_W = _SEQ // _NW

_CH = 64
_NCH = _B_PER_W // _CH

_mesh = plsc.VectorSubcoreMesh(core_axis_name="c", subcore_axis_name="s")


@functools.partial(
    pl.kernel,
    mesh=_mesh,
    out_type=jax.ShapeDtypeStruct((_SEQ, _D), jnp.float32),
    scratch_types=[
        pltpu.VMEM((_NCH, _CH), jnp.int32),
        pltpu.VMEM((_CH, _D), jnp.float32),
        pltpu.VMEM((_CH, _D), jnp.float32),
        pltpu.SemaphoreType.DMA,
        pltpu.SemaphoreType.DMA,
        pltpu.SemaphoreType.DMA,
        pltpu.SemaphoreType.DMA,
    ],
)
def _sc_gather(table_hbm, idx_hbm, out_hbm, idx_v, buf0, buf1,
               gsem0, gsem1, wsem0, wsem1):
    wid = lax.axis_index("s") * _NC + lax.axis_index("c")
    base = wid * _B_PER_W
    pltpu.sync_copy(idx_hbm.at[wid], idx_v)
    bufs = (buf0, buf1)
    gsems = (gsem0, gsem1)
    wsems = (wsem0, wsem1)

    def start_gather(c):
        return pltpu.async_copy(table_hbm.at[idx_v.at[c]], bufs[c % 2],
                                gsems[c % 2])

    g = [start_gather(0), start_gather(1)]
    for c in range(_NCH):
        b = c % 2
        g[b].wait()
        w = pltpu.async_copy(bufs[b], out_hbm.at[pl.ds(base + c * _CH, _CH)],
                             wsems[b])
        if c + 2 < _NCH:
            w.wait()
            g[b] = start_gather(c + 2)
        else:
            w.wait()


_BLK = 512
_LOG1E4_2_OVER_D = 2.0 * math.log(10000.0) / _D
_NJ2 = 2 * ((_D - 1) // 2)


def _add_pos_body(emb_ref, out_ref):
    i = pl.program_id(0)
    row = (jnp.float32(i * _BLK)
           + lax.broadcasted_iota(jnp.int32, (_BLK, _D), 0).astype(jnp.float32))
    col = lax.broadcasted_iota(jnp.int32, (_BLK, _D), 1)
    j = jnp.floor_divide(col, 2).astype(jnp.float32)
    w = jnp.exp(j * (-_LOG1E4_2_OVER_D))
    phase = jnp.where(col % 2 == 0, 0.0, jnp.float32(math.pi / 2))
    pos = jnp.sin(row * w + phase)
    pos = jnp.where(col >= _NJ2, 0.0, pos)
    out_ref[...] = emb_ref[...] + pos


def _add_pos(emb):
    return pl.pallas_call(
        _add_pos_body,
        grid=(_SEQ // _BLK,),
        in_specs=[pl.BlockSpec((_BLK, _D), lambda i: (i, 0))],
        out_specs=pl.BlockSpec((_BLK, _D), lambda i: (i, 0)),
        out_shape=jax.ShapeDtypeStruct((_SEQ, _D), jnp.float32),
    )(emb)


def kernel(x, table):
    idx = x.astype(jnp.int32).reshape(_NW, _NCH, _CH)
    emb = _sc_gather(table, idx)
    return _add_pos(emb)
